# baseline (speedup 1.0000x reference)
import jax
import jax.numpy as jnp
from jax.experimental import pallas as pl
from jax.experimental.pallas import tpu as pltpu

COMMITMENT_COST = 0.25
EPSILON = 1e-10


def _vq_block_kernel(x_ref, w_ref, dist_ref, enc_ref, idx_ref, q_ref,
                     loss_ref, perp_ref, w2_ref, counts_ref, ssq_ref):
    step = pl.program_id(0)
    nsteps = pl.num_programs(0)
    xb = x_ref[...]
    wm = w_ref[...]
    bm = xb.shape[0]
    n = wm.shape[1]

    @pl.when(step == 0)
    def _w2():
        w2_ref[...] = jnp.sum(wm * wm, axis=0, keepdims=True)

    x2 = jnp.sum(xb * xb, axis=1, keepdims=True)
    mm2 = jnp.dot(xb * (-2.0), wm,
                  preferred_element_type=jnp.float32)
    d = (x2 + mm2) + w2_ref[...]
    dist_ref[...] = d

    mn = jnp.min(d, axis=1, keepdims=True)
    iota = jax.lax.broadcasted_iota(jnp.int32, (bm, n), 1)
    idx = jnp.min(jnp.where(d == mn, iota, n), axis=1, keepdims=True)
    idx_ref[...] = idx

    enc = (iota == idx).astype(jnp.float32)
    enc_ref[...] = enc

    q = jax.lax.dot_general(enc, wm, (((1,), (1,)), ((), ())),
                            preferred_element_type=jnp.float32)
    q_ref[...] = q

    ssq = jnp.sum(mn).reshape(1, 1)
    cnt = jnp.sum(enc, axis=0, keepdims=True)

    @pl.when(step == 0)
    def _init():
        counts_ref[...] = cnt
        ssq_ref[...] = ssq

    @pl.when(step > 0)
    def _acc():
        counts_ref[...] += cnt
        ssq_ref[...] += ssq

    @pl.when(step == nsteps - 1)
    def _fin():
        total = jnp.float32(bm) * nsteps
        avg = counts_ref[...] / total
        ent = -jnp.sum(avg * jnp.log(avg + EPSILON))
        perp_ref[...] = jnp.exp(ent).reshape(1, 1)
        scale = (1.0 + COMMITMENT_COST) / (total * xb.shape[1])
        loss_ref[...] = ssq_ref[...] * scale


def kernel(x, w):
    k = w.shape[0]
    n = w.shape[1]
    xf = x.reshape(-1, k)
    m = xf.shape[0]
    bm = 256 if m % 256 == 0 else m
    grid = m // bm

    out_types = (
        jax.ShapeDtypeStruct((m, n), jnp.float32),
        jax.ShapeDtypeStruct((m, n), jnp.float32),
        jax.ShapeDtypeStruct((m, 1), jnp.int32),
        jax.ShapeDtypeStruct((m, k), jnp.float32),
        jax.ShapeDtypeStruct((1, 1), jnp.float32),
        jax.ShapeDtypeStruct((1, 1), jnp.float32),
    )
    dist, enc, idx, q, loss, perp = pl.pallas_call(
        _vq_block_kernel,
        grid=(grid,),
        in_specs=[
            pl.BlockSpec((bm, k), lambda i: (i, 0)),
            pl.BlockSpec((k, n), lambda i: (0, 0)),
        ],
        out_specs=(
            pl.BlockSpec((bm, n), lambda i: (i, 0)),
            pl.BlockSpec((bm, n), lambda i: (i, 0)),
            pl.BlockSpec((bm, 1), lambda i: (i, 0)),
            pl.BlockSpec((bm, k), lambda i: (i, 0)),
            pl.BlockSpec((1, 1), lambda i: (0, 0)),
            pl.BlockSpec((1, 1), lambda i: (0, 0)),
        ),
        out_shape=out_types,
        scratch_shapes=[
            pltpu.VMEM((1, n), jnp.float32),
            pltpu.VMEM((1, n), jnp.float32),
            pltpu.VMEM((1, 1), jnp.float32),
        ],
    )(xf, w)

    quantized_st = q.reshape(x.shape)
    encoding_indices = idx.reshape(x.shape[:-1])
    return (quantized_st, loss[0, 0], perp[0, 0], enc, encoding_indices, dist)


# speedup vs baseline: 1.2147x; 1.2147x over previous
import jax
import jax.numpy as jnp
from jax.experimental import pallas as pl
from jax.experimental.pallas import tpu as pltpu

COMMITMENT_COST = 0.25
EPSILON = 1e-10


def _vq_block_kernel(x_ref, w_ref, dist_ref, enc_ref, idx_ref, q_ref,
                     loss_ref, perp_ref, wneg2_ref, w2_ref, counts_ref,
                     ssq_ref):
    step = pl.program_id(0)
    nsteps = pl.num_programs(0)
    xb = x_ref[...]
    bm = xb.shape[0]
    n = w_ref.shape[1]

    @pl.when(step == 0)
    def _prep():
        wm0 = w_ref[...]
        wneg2_ref[...] = wm0 * (-2.0)
        w2_ref[...] = jnp.sum(wm0 * wm0, axis=0, keepdims=True)

    x2 = jnp.sum(xb * xb, axis=1, keepdims=True)
    mm2 = jnp.dot(xb, wneg2_ref[...],
                  preferred_element_type=jnp.float32)
    d = (x2 + mm2) + w2_ref[...]
    dist_ref[...] = d

    idx = jnp.argmin(d, axis=1).reshape(bm, 1)
    idx_ref[...] = idx

    iota = jax.lax.broadcasted_iota(jnp.int32, (bm, n), 1)
    enc = (iota == idx).astype(jnp.float32)
    enc_ref[...] = enc

    q = jax.lax.dot_general(enc, w_ref[...], (((1,), (1,)), ((), ())),
                            preferred_element_type=jnp.float32)
    q_ref[...] = q

    diff = q - xb
    ssq = jnp.sum(diff * diff).reshape(1, 1)
    ones_row = jnp.full((1, bm), 1.0, jnp.float32)
    cnt = jnp.dot(ones_row, enc,
                  preferred_element_type=jnp.float32)

    @pl.when(step == 0)
    def _init():
        counts_ref[...] = cnt
        ssq_ref[...] = ssq

    @pl.when(step > 0)
    def _acc():
        counts_ref[...] += cnt
        ssq_ref[...] += ssq

    @pl.when(step == nsteps - 1)
    def _fin():
        total = jnp.float32(bm) * nsteps
        avg = counts_ref[...] / total
        ent = -jnp.sum(avg * jnp.log(avg + EPSILON))
        perp_ref[...] = jnp.exp(ent).reshape(1, 1)
        scale = (1.0 + COMMITMENT_COST) / (total * xb.shape[1])
        loss_ref[...] = ssq_ref[...] * scale


def kernel(x, w):
    k = w.shape[0]
    n = w.shape[1]
    xf = x.reshape(-1, k)
    m = xf.shape[0]
    bm = 256 if m % 256 == 0 else m
    grid = m // bm

    out_types = (
        jax.ShapeDtypeStruct((m, n), jnp.float32),
        jax.ShapeDtypeStruct((m, n), jnp.float32),
        jax.ShapeDtypeStruct((m, 1), jnp.int32),
        jax.ShapeDtypeStruct((m, k), jnp.float32),
        jax.ShapeDtypeStruct((1, 1), jnp.float32),
        jax.ShapeDtypeStruct((1, 1), jnp.float32),
    )
    dist, enc, idx, q, loss, perp = pl.pallas_call(
        _vq_block_kernel,
        grid=(grid,),
        in_specs=[
            pl.BlockSpec((bm, k), lambda i: (i, 0)),
            pl.BlockSpec((k, n), lambda i: (0, 0)),
        ],
        out_specs=(
            pl.BlockSpec((bm, n), lambda i: (i, 0)),
            pl.BlockSpec((bm, n), lambda i: (i, 0)),
            pl.BlockSpec((bm, 1), lambda i: (i, 0)),
            pl.BlockSpec((bm, k), lambda i: (i, 0)),
            pl.BlockSpec((1, 1), lambda i: (0, 0)),
            pl.BlockSpec((1, 1), lambda i: (0, 0)),
        ),
        out_shape=out_types,
        scratch_shapes=[
            pltpu.VMEM((k, n), jnp.float32),
            pltpu.VMEM((1, n), jnp.float32),
            pltpu.VMEM((1, n), jnp.float32),
            pltpu.VMEM((1, 1), jnp.float32),
        ],
    )(xf, w)

    quantized_st = q.reshape(x.shape)
    encoding_indices = idx.reshape(x.shape[:-1])
    return (quantized_st, loss[0, 0], perp[0, 0], enc, encoding_indices, dist)
